# Initial kernel scaffold; baseline (speedup 1.0000x reference)
import jax
import jax.numpy as jnp
from jax.experimental import pallas as pl


def kernel(x_user, x_product, x_category, x_brand, x_shop, type_emb, W1_0, b1_0, W2_0, b2_0, W1_1, b1_1, W2_1, b2_1, W1_2, b1_2, W2_2, b2_2, Wout, bout, edge_index):
    raise NotImplementedError("write your pallas kernel here")


# speedup vs baseline: 1.7130x; 1.7130x over previous
import functools

import jax
import jax.numpy as jnp
from jax import lax
from jax.experimental import pallas as pl
from jax.experimental.pallas import tpu as pltpu
from jax.experimental.pallas import tpu_sc as plsc

N = 10000
NPT = 2000
E = 160000
H = 512
NC = 2
NS = 16
B = 128
CW = 64
NBLK = 80
E_PAD = NS * NBLK * B
RPAD = 10240
STRIPE = RPAD // NS
RB = 1000

_PREC = lax.Precision.HIGHEST


def _make_segsum(nch):
    npc = nch // NC
    mesh = plsc.VectorSubcoreMesh(core_axis_name="c", subcore_axis_name="s",
                                  num_cores=NC, num_subcores=NS)

    @functools.partial(
        pl.kernel,
        out_type=jax.ShapeDtypeStruct((RPAD, nch, CW), jnp.float32),
        mesh=mesh,
        compiler_params=pltpu.CompilerParams(use_tc_tiling_on_sc=False),
        scratch_types=[
            pltpu.VMEM((NBLK, B), jnp.int32),
            pltpu.VMEM((NBLK, B), jnp.int32),
            pltpu.VMEM((2, B, CW), jnp.float32),
            pltpu.VMEM_SHARED((RPAD, CW), jnp.float32),
            pltpu.SemaphoreType.DMA,
            pltpu.SemaphoreType.DMA,
        ],
    )
    def seg(h_hbm, srcoff_hbm, dst_hbm, zeros_hbm, out_hbm,
            src_v, dst_v, rows_v, acc, gsem0, gsem1):
        cid = lax.axis_index("c")
        sid = lax.axis_index("s")
        pltpu.sync_copy(dst_hbm.at[sid], dst_v)
        gsems = (gsem0, gsem1)
        for k in range(npc):
            chunk = k * NC + cid
            pltpu.sync_copy(srcoff_hbm.at[chunk, sid], src_v)
            pltpu.sync_copy(zeros_hbm, acc.at[pl.ds(sid * STRIPE, STRIPE)])
            plsc.subcore_barrier()
            for b in range(2):
                pltpu.async_copy(h_hbm.at[src_v.at[b]], rows_v.at[b], gsems[b])

            @pl.loop(0, NBLK, step=2)
            def _(g):
                for b in range(2):
                    blk = g + b
                    pltpu.make_async_copy(
                        h_hbm.at[src_v.at[blk]], rows_v.at[b], gsems[b]).wait()
                    pltpu.sync_copy(rows_v.at[b], acc.at[dst_v.at[blk]],
                                    add=True)

                    @pl.when(blk + 2 < NBLK)
                    def _():
                        pltpu.async_copy(h_hbm.at[src_v.at[blk + 2]],
                                         rows_v.at[b], gsems[b])
            plsc.subcore_barrier()
            pltpu.sync_copy(acc.at[pl.ds(sid * STRIPE, STRIPE)],
                            out_hbm.at[pl.ds(sid * STRIPE, STRIPE), chunk])
            plsc.subcore_barrier()

    return seg


def _mlp1(din, nblocks, roff):
    def body(h_ref, agg_ref, w_ref, b_ref, out_ref):
        z = h_ref[...] + agg_ref[...]
        out_ref[...] = jnp.maximum(
            jnp.dot(z, w_ref[...], precision=_PREC) + b_ref[...], 0.0)

    return pl.pallas_call(
        body,
        grid=(nblocks,),
        in_specs=[
            pl.BlockSpec((RB, din), lambda i: (i + roff, 0)),
            pl.BlockSpec((RB, din), lambda i: (i + roff, 0)),
            pl.BlockSpec((din, H), lambda i: (0, 0)),
            pl.BlockSpec((1, H), lambda i: (0, 0)),
        ],
        out_specs=pl.BlockSpec((RB, H), lambda i: (i, 0)),
        out_shape=jax.ShapeDtypeStruct((nblocks * RB, H), jnp.float32),
    )


def _mlp2():
    def body(u_ref, w_ref, b_ref, out_ref):
        out_ref[...] = jnp.maximum(
            jnp.dot(u_ref[...], w_ref[...], precision=_PREC) + b_ref[...], 0.0)

    return pl.pallas_call(
        body,
        grid=(N // RB,),
        in_specs=[
            pl.BlockSpec((RB, H), lambda i: (i, 0)),
            pl.BlockSpec((H, H), lambda i: (0, 0)),
            pl.BlockSpec((1, H), lambda i: (0, 0)),
        ],
        out_specs=pl.BlockSpec((RB, H), lambda i: (i, 0)),
        out_shape=jax.ShapeDtypeStruct((N, H), jnp.float32),
    )


def _mlp2_head():
    def body(u_ref, w_ref, b_ref, wo_ref, out_ref):
        v = jnp.maximum(jnp.dot(u_ref[...], w_ref[...], precision=_PREC)
                        + b_ref[...], 0.0)
        out_ref[...] = jnp.dot(v, wo_ref[...], precision=_PREC)

    return pl.pallas_call(
        body,
        grid=(NPT // RB,),
        in_specs=[
            pl.BlockSpec((RB, H), lambda i: (i, 0)),
            pl.BlockSpec((H, H), lambda i: (0, 0)),
            pl.BlockSpec((1, H), lambda i: (0, 0)),
            pl.BlockSpec((H, 128), lambda i: (0, 0)),
        ],
        out_specs=pl.BlockSpec((RB, 128), lambda i: (i, 0)),
        out_shape=jax.ShapeDtypeStruct((NPT, 128), jnp.float32),
    )


def kernel(x_user, x_product, x_category, x_brand, x_shop, type_emb,
           W1_0, b1_0, W2_0, b2_0, W1_1, b1_1, W2_1, b2_1, W1_2, b1_2,
           W2_2, b2_2, Wout, bout, edge_index):
    f32 = jnp.float32
    W0 = 384
    NCH0 = W0 // CW
    NCH = H // CW
    x_all = jnp.concatenate([x_user, x_product, x_category, x_brand, x_shop], 0)
    emb = jnp.repeat(type_emb, NPT, axis=0)
    h0 = jnp.concatenate([x_all, emb, jnp.zeros((N, 120), f32)], 1)
    W1p0 = jnp.concatenate([W1_0, jnp.zeros((120, H), f32)], 0)
    Wop = jnp.concatenate([Wout, jnp.zeros((H, 127), f32)], 1)

    src = edge_index[0]
    dst = edge_index[1]
    pad = E_PAD - E
    src_p = jnp.concatenate([src, jnp.zeros((pad,), jnp.int32)])
    trash = N + (jnp.arange(pad, dtype=jnp.int32) % (RPAD - N))
    dst_p = jnp.concatenate([dst, trash])
    src_w = src_p.reshape(NS, NBLK, B)
    dst_w = dst_p.reshape(NS, NBLK, B)
    off0 = jnp.arange(NCH0, dtype=jnp.int32)[:, None, None, None]
    off = jnp.arange(NCH, dtype=jnp.int32)[:, None, None, None]
    srcoff0 = src_w[None] * NCH0 + off0
    srcoff = src_w[None] * NCH + off
    zeros_t = jnp.zeros((STRIPE, CW), f32)

    seg0 = _make_segsum(NCH0)
    seg = _make_segsum(NCH)
    mlp1_0 = _mlp1(W0, N // RB, 0)
    mlp1_1 = _mlp1(H, N // RB, 0)
    mlp1_2 = _mlp1(H, NPT // RB, NPT // RB)
    mlp2 = _mlp2()
    mlp2h = _mlp2_head()

    agg0 = seg0(h0.reshape(NCH0 * N, CW), srcoff0, dst_w, zeros_t)
    u0 = mlp1_0(h0, agg0.reshape(RPAD, W0), W1p0, b1_0[None])
    h1 = mlp2(u0, W2_0, b2_0[None])
    agg1 = seg(h1.reshape(NCH * N, CW), srcoff, dst_w, zeros_t)
    u1 = mlp1_1(h1, agg1.reshape(RPAD, H), W1_1, b1_1[None])
    h2 = mlp2(u1, W2_1, b2_1[None])
    agg2 = seg(h2.reshape(NCH * N, CW), srcoff, dst_w, zeros_t)
    u2 = mlp1_2(h2, agg2.reshape(RPAD, H), W1_2, b1_2[None])
    y = mlp2h(u2, W2_2, b2_2[None], Wop)
    return y[:, 0] + bout[0]


# speedup vs baseline: 1.7172x; 1.0025x over previous
import functools

import jax
import jax.numpy as jnp
from jax import lax
from jax.experimental import pallas as pl
from jax.experimental.pallas import tpu as pltpu
from jax.experimental.pallas import tpu_sc as plsc

N = 10000
NPT = 2000
E = 160000
H = 512
NC = 2
NS = 16
B = 128
CW = 64
NBLK = 80
LOOK = 1
NBUF = 2 * LOOK
E_PAD = NS * NBLK * B
RPAD = 10240
STRIPE = RPAD // NS
RB = 1000

_PREC = lax.Precision.HIGHEST


def _make_segsum(nch):
    npc = nch // NC
    mesh = plsc.VectorSubcoreMesh(core_axis_name="c", subcore_axis_name="s",
                                  num_cores=NC, num_subcores=NS)

    @functools.partial(
        pl.kernel,
        out_type=jax.ShapeDtypeStruct((RPAD, nch, CW), jnp.float32),
        mesh=mesh,
        compiler_params=pltpu.CompilerParams(use_tc_tiling_on_sc=False),
        scratch_types=[
            pltpu.VMEM((NBLK, B), jnp.int32),
            pltpu.VMEM((NBLK, B), jnp.int32),
            pltpu.VMEM((NBUF, B, CW), jnp.float32),
            pltpu.VMEM_SHARED((RPAD, CW), jnp.float32),
            [pltpu.SemaphoreType.DMA] * NBUF,
            [pltpu.SemaphoreType.DMA] * NBUF,
        ],
    )
    def seg(h_hbm, srcoff_hbm, dst_hbm, zeros_hbm, out_hbm,
            src_v, dst_v, rows_v, acc, gsems, ssems):
        cid = lax.axis_index("c")
        sid = lax.axis_index("s")
        pltpu.sync_copy(dst_hbm.at[sid], dst_v)
        for k in range(npc):
            chunk = k * NC + cid
            pltpu.sync_copy(srcoff_hbm.at[chunk, sid], src_v)
            for b in range(LOOK):
                pltpu.async_copy(h_hbm.at[src_v.at[b]], rows_v.at[b], gsems[b])
            pltpu.sync_copy(zeros_hbm, acc.at[pl.ds(sid * STRIPE, STRIPE)])
            plsc.subcore_barrier()

            @pl.loop(0, NBLK, step=NBUF)
            def _(g):
                for i in range(NBUF):
                    blk = g + i
                    bi = i
                    bg = (i + LOOK) % NBUF

                    @pl.when((blk >= LOOK) & (blk + LOOK < NBLK))
                    def _():
                        pltpu.make_async_copy(
                            rows_v.at[bg], acc.at[dst_v.at[blk]],
                            ssems[bg]).wait()

                    @pl.when(blk + LOOK < NBLK)
                    def _():
                        pltpu.async_copy(h_hbm.at[src_v.at[blk + LOOK]],
                                         rows_v.at[bg], gsems[bg])
                    pltpu.make_async_copy(
                        h_hbm.at[src_v.at[blk]], rows_v.at[bi],
                        gsems[bi]).wait()
                    pltpu.async_copy(rows_v.at[bi], acc.at[dst_v.at[blk]],
                                     ssems[bi], add=True)
            for b in range(NBUF):
                pltpu.make_async_copy(rows_v.at[b], acc.at[dst_v.at[0]],
                                      ssems[b]).wait()
            plsc.subcore_barrier()
            pltpu.sync_copy(acc.at[pl.ds(sid * STRIPE, STRIPE)],
                            out_hbm.at[pl.ds(sid * STRIPE, STRIPE), chunk])
            plsc.subcore_barrier()

    return seg


def _mlp1(din, nblocks, roff):
    def body(h_ref, agg_ref, w_ref, b_ref, out_ref):
        z = h_ref[...] + agg_ref[...]
        out_ref[...] = jnp.maximum(
            jnp.dot(z, w_ref[...], precision=_PREC) + b_ref[...], 0.0)

    return pl.pallas_call(
        body,
        grid=(nblocks,),
        in_specs=[
            pl.BlockSpec((RB, din), lambda i: (i + roff, 0)),
            pl.BlockSpec((RB, din), lambda i: (i + roff, 0)),
            pl.BlockSpec((din, H), lambda i: (0, 0)),
            pl.BlockSpec((1, H), lambda i: (0, 0)),
        ],
        out_specs=pl.BlockSpec((RB, H), lambda i: (i, 0)),
        out_shape=jax.ShapeDtypeStruct((nblocks * RB, H), jnp.float32),
    )


def _mlp2():
    def body(u_ref, w_ref, b_ref, out_ref):
        out_ref[...] = jnp.maximum(
            jnp.dot(u_ref[...], w_ref[...], precision=_PREC) + b_ref[...], 0.0)

    return pl.pallas_call(
        body,
        grid=(N // RB,),
        in_specs=[
            pl.BlockSpec((RB, H), lambda i: (i, 0)),
            pl.BlockSpec((H, H), lambda i: (0, 0)),
            pl.BlockSpec((1, H), lambda i: (0, 0)),
        ],
        out_specs=pl.BlockSpec((RB, H), lambda i: (i, 0)),
        out_shape=jax.ShapeDtypeStruct((N, H), jnp.float32),
    )


def _mlp2_head():
    def body(u_ref, w_ref, b_ref, wo_ref, out_ref):
        v = jnp.maximum(jnp.dot(u_ref[...], w_ref[...], precision=_PREC)
                        + b_ref[...], 0.0)
        out_ref[...] = jnp.dot(v, wo_ref[...], precision=_PREC)

    return pl.pallas_call(
        body,
        grid=(NPT // RB,),
        in_specs=[
            pl.BlockSpec((RB, H), lambda i: (i, 0)),
            pl.BlockSpec((H, H), lambda i: (0, 0)),
            pl.BlockSpec((1, H), lambda i: (0, 0)),
            pl.BlockSpec((H, 128), lambda i: (0, 0)),
        ],
        out_specs=pl.BlockSpec((RB, 128), lambda i: (i, 0)),
        out_shape=jax.ShapeDtypeStruct((NPT, 128), jnp.float32),
    )


def kernel(x_user, x_product, x_category, x_brand, x_shop, type_emb,
           W1_0, b1_0, W2_0, b2_0, W1_1, b1_1, W2_1, b2_1, W1_2, b1_2,
           W2_2, b2_2, Wout, bout, edge_index):
    f32 = jnp.float32
    W0 = 384
    NCH0 = W0 // CW
    NCH = H // CW
    x_all = jnp.concatenate([x_user, x_product, x_category, x_brand, x_shop], 0)
    emb = jnp.repeat(type_emb, NPT, axis=0)
    h0 = jnp.concatenate([x_all, emb, jnp.zeros((N, 120), f32)], 1)
    W1p0 = jnp.concatenate([W1_0, jnp.zeros((120, H), f32)], 0)
    Wop = jnp.concatenate([Wout, jnp.zeros((H, 127), f32)], 1)

    src = edge_index[0]
    dst = edge_index[1]
    pad = E_PAD - E
    src_p = jnp.concatenate([src, jnp.zeros((pad,), jnp.int32)])
    trash = N + (jnp.arange(pad, dtype=jnp.int32) % (RPAD - N))
    dst_p = jnp.concatenate([dst, trash])
    src_w = src_p.reshape(NS, NBLK, B)
    dst_w = dst_p.reshape(NS, NBLK, B)
    off0 = jnp.arange(NCH0, dtype=jnp.int32)[:, None, None, None]
    off = jnp.arange(NCH, dtype=jnp.int32)[:, None, None, None]
    srcoff0 = src_w[None] * NCH0 + off0
    srcoff = src_w[None] * NCH + off
    zeros_t = jnp.zeros((STRIPE, CW), f32)

    seg0 = _make_segsum(NCH0)
    seg = _make_segsum(NCH)
    mlp1_0 = _mlp1(W0, N // RB, 0)
    mlp1_1 = _mlp1(H, N // RB, 0)
    mlp1_2 = _mlp1(H, NPT // RB, NPT // RB)
    mlp2 = _mlp2()
    mlp2h = _mlp2_head()

    agg0 = seg0(h0.reshape(NCH0 * N, CW), srcoff0, dst_w, zeros_t)
    u0 = mlp1_0(h0, agg0.reshape(RPAD, W0), W1p0, b1_0[None])
    h1 = mlp2(u0, W2_0, b2_0[None])
    agg1 = seg(h1.reshape(NCH * N, CW), srcoff, dst_w, zeros_t)
    u1 = mlp1_1(h1, agg1.reshape(RPAD, H), W1_1, b1_1[None])
    h2 = mlp2(u1, W2_1, b2_1[None])
    agg2 = seg(h2.reshape(NCH * N, CW), srcoff, dst_w, zeros_t)
    u2 = mlp1_2(h2, agg2.reshape(RPAD, H), W1_2, b1_2[None])
    y = mlp2h(u2, W2_2, b2_2[None], Wop)
    return y[:, 0] + bout[0]


# speedup vs baseline: 2.1894x; 1.2749x over previous
import functools

import jax
import jax.numpy as jnp
from jax import lax
from jax.experimental import pallas as pl
from jax.experimental.pallas import tpu as pltpu
from jax.experimental.pallas import tpu_sc as plsc

N = 10000
NPT = 2000
E = 160000
H = 512
NC = 2
NS = 16
B = 128
CW = 128
NBLK = 80
NBLKH = NBLK // 2
LOOK = 1
NBUF = 2 * LOOK
E_PAD = NS * NBLK * B
RPAD = 10240
STRIPE = RPAD // NS
RB = 1000

_PREC = lax.Precision.HIGHEST


def _make_segsum(nch):
    npc = (nch + NC - 1) // NC
    mesh = plsc.VectorSubcoreMesh(core_axis_name="c", subcore_axis_name="s",
                                  num_cores=NC, num_subcores=NS)

    @functools.partial(
        pl.kernel,
        out_type=jax.ShapeDtypeStruct((RPAD, nch, CW), jnp.float32),
        mesh=mesh,
        compiler_params=pltpu.CompilerParams(use_tc_tiling_on_sc=False),
        scratch_types=[
            pltpu.VMEM((NBLKH, B), jnp.int32),
            pltpu.VMEM((NBLKH, B), jnp.int32),
            pltpu.VMEM((NBUF, B, CW), jnp.float32),
            pltpu.VMEM_SHARED((RPAD, CW), jnp.float32),
            [pltpu.SemaphoreType.DMA] * NBUF,
            [pltpu.SemaphoreType.DMA] * NBUF,
        ],
    )
    def seg(h_hbm, srcoff_hbm, dst_hbm, zeros_hbm, out_hbm,
            src_v, dst_v, rows_v, acc, gsems, ssems):
        cid = lax.axis_index("c")
        sid = lax.axis_index("s")

        def _chunk_body(chunk):
            for half in range(2):
                base = half * NBLKH
                pltpu.sync_copy(srcoff_hbm.at[chunk, sid,
                                              pl.ds(base, NBLKH)], src_v)
                pltpu.sync_copy(dst_hbm.at[sid, pl.ds(base, NBLKH)], dst_v)
                for b in range(LOOK):
                    pltpu.async_copy(h_hbm.at[src_v.at[b]], rows_v.at[b],
                                     gsems[b])
                if half == 0:
                    pltpu.sync_copy(zeros_hbm,
                                    acc.at[pl.ds(sid * STRIPE, STRIPE)])
                    plsc.subcore_barrier()

                @pl.loop(0, NBLKH, step=NBUF)
                def _(g):
                    for i in range(NBUF):
                        blk = g + i
                        bi = i
                        bg = (i + LOOK) % NBUF

                        @pl.when((blk >= LOOK) & (blk + LOOK < NBLKH))
                        def _():
                            pltpu.make_async_copy(
                                rows_v.at[bg], acc.at[dst_v.at[blk]],
                                ssems[bg]).wait()

                        @pl.when(blk + LOOK < NBLKH)
                        def _():
                            pltpu.async_copy(h_hbm.at[src_v.at[blk + LOOK]],
                                             rows_v.at[bg], gsems[bg])
                        pltpu.make_async_copy(
                            h_hbm.at[src_v.at[blk]], rows_v.at[bi],
                            gsems[bi]).wait()
                        pltpu.async_copy(rows_v.at[bi], acc.at[dst_v.at[blk]],
                                         ssems[bi], add=True)
                for b in range(NBUF):
                    pltpu.make_async_copy(rows_v.at[b], acc.at[dst_v.at[0]],
                                          ssems[b]).wait()
            plsc.subcore_barrier()
            pltpu.sync_copy(acc.at[pl.ds(sid * STRIPE, STRIPE)],
                            out_hbm.at[pl.ds(sid * STRIPE, STRIPE), chunk])
            plsc.subcore_barrier()

        for k in range(npc):
            chunk = k * NC + cid

            @pl.when(chunk < nch)
            def _chunk_round(chunk=chunk):
                _chunk_body(chunk)

    return seg


def _mlp1(din, nblocks, roff):
    def body(h_ref, agg_ref, w_ref, b_ref, out_ref):
        z = h_ref[...] + agg_ref[...]
        out_ref[...] = jnp.maximum(
            jnp.dot(z, w_ref[...], precision=_PREC) + b_ref[...], 0.0)

    return pl.pallas_call(
        body,
        grid=(nblocks,),
        in_specs=[
            pl.BlockSpec((RB, din), lambda i: (i + roff, 0)),
            pl.BlockSpec((RB, din), lambda i: (i + roff, 0)),
            pl.BlockSpec((din, H), lambda i: (0, 0)),
            pl.BlockSpec((1, H), lambda i: (0, 0)),
        ],
        out_specs=pl.BlockSpec((RB, H), lambda i: (i, 0)),
        out_shape=jax.ShapeDtypeStruct((nblocks * RB, H), jnp.float32),
    )


def _mlp2():
    def body(u_ref, w_ref, b_ref, out_ref):
        out_ref[...] = jnp.maximum(
            jnp.dot(u_ref[...], w_ref[...], precision=_PREC) + b_ref[...], 0.0)

    return pl.pallas_call(
        body,
        grid=(N // RB,),
        in_specs=[
            pl.BlockSpec((RB, H), lambda i: (i, 0)),
            pl.BlockSpec((H, H), lambda i: (0, 0)),
            pl.BlockSpec((1, H), lambda i: (0, 0)),
        ],
        out_specs=pl.BlockSpec((RB, H), lambda i: (i, 0)),
        out_shape=jax.ShapeDtypeStruct((N, H), jnp.float32),
    )


def _mlp2_head():
    def body(u_ref, w_ref, b_ref, wo_ref, out_ref):
        v = jnp.maximum(jnp.dot(u_ref[...], w_ref[...], precision=_PREC)
                        + b_ref[...], 0.0)
        out_ref[...] = jnp.dot(v, wo_ref[...], precision=_PREC)

    return pl.pallas_call(
        body,
        grid=(NPT // RB,),
        in_specs=[
            pl.BlockSpec((RB, H), lambda i: (i, 0)),
            pl.BlockSpec((H, H), lambda i: (0, 0)),
            pl.BlockSpec((1, H), lambda i: (0, 0)),
            pl.BlockSpec((H, 128), lambda i: (0, 0)),
        ],
        out_specs=pl.BlockSpec((RB, 128), lambda i: (i, 0)),
        out_shape=jax.ShapeDtypeStruct((NPT, 128), jnp.float32),
    )


def kernel(x_user, x_product, x_category, x_brand, x_shop, type_emb,
           W1_0, b1_0, W2_0, b2_0, W1_1, b1_1, W2_1, b2_1, W1_2, b1_2,
           W2_2, b2_2, Wout, bout, edge_index):
    f32 = jnp.float32
    W0 = 384
    NCH0 = W0 // CW
    NCH = H // CW
    x_all = jnp.concatenate([x_user, x_product, x_category, x_brand, x_shop], 0)
    emb = jnp.repeat(type_emb, NPT, axis=0)
    h0 = jnp.concatenate([x_all, emb, jnp.zeros((N, 120), f32)], 1)
    W1p0 = jnp.concatenate([W1_0, jnp.zeros((120, H), f32)], 0)
    Wop = jnp.concatenate([Wout, jnp.zeros((H, 127), f32)], 1)

    src = edge_index[0]
    dst = edge_index[1]
    pad = E_PAD - E
    src_p = jnp.concatenate([src, jnp.zeros((pad,), jnp.int32)])
    trash = N + (jnp.arange(pad, dtype=jnp.int32) % (RPAD - N))
    dst_p = jnp.concatenate([dst, trash])
    src_w = src_p.reshape(NS, NBLK, B)
    dst_w = dst_p.reshape(NS, NBLK, B)
    off0 = jnp.arange(NCH0, dtype=jnp.int32)[:, None, None, None]
    off = jnp.arange(NCH, dtype=jnp.int32)[:, None, None, None]
    srcoff0 = src_w[None] * NCH0 + off0
    srcoff = src_w[None] * NCH + off
    zeros_t = jnp.zeros((STRIPE, CW), f32)

    seg0 = _make_segsum(NCH0)
    seg = _make_segsum(NCH)
    mlp1_0 = _mlp1(W0, N // RB, 0)
    mlp1_1 = _mlp1(H, N // RB, 0)
    mlp1_2 = _mlp1(H, NPT // RB, NPT // RB)
    mlp2 = _mlp2()
    mlp2h = _mlp2_head()

    agg0 = seg0(h0.reshape(NCH0 * N, CW), srcoff0, dst_w, zeros_t)
    u0 = mlp1_0(h0, agg0.reshape(RPAD, W0), W1p0, b1_0[None])
    h1 = mlp2(u0, W2_0, b2_0[None])
    agg1 = seg(h1.reshape(NCH * N, CW), srcoff, dst_w, zeros_t)
    u1 = mlp1_1(h1, agg1.reshape(RPAD, H), W1_1, b1_1[None])
    h2 = mlp2(u1, W2_1, b2_1[None])
    agg2 = seg(h2.reshape(NCH * N, CW), srcoff, dst_w, zeros_t)
    u2 = mlp1_2(h2, agg2.reshape(RPAD, H), W1_2, b1_2[None])
    y = mlp2h(u2, W2_2, b2_2[None], Wop)
    return y[:, 0] + bout[0]
